# Initial kernel scaffold; baseline (speedup 1.0000x reference)
import jax
import jax.numpy as jnp
from jax.experimental import pallas as pl


def kernel(variable_states, edge_probs, intervention_indices, intervention_values, W1, b1, W2, b2, G1, g1, G2, g2):
    raise NotImplementedError("write your pallas kernel here")


# speedup vs baseline: 2.9369x; 2.9369x over previous
import functools

import jax
import jax.numpy as jnp
from jax.experimental import pallas as pl
from jax.experimental.pallas import tpu as pltpu

_INV_SQRT2 = 0.7071067811865476


def _gelu(x):
    return 0.5 * x * (1.0 + jax.lax.erf(x * _INV_SQRT2))


def _dot_t(x, w):
    return jax.lax.dot_general(
        x, w, dimension_numbers=(((1,), (1,)), ((), ())),
        preferred_element_type=jnp.float32)


def _body(idx_ref, vs_ref, vals_ref, W1_ref, b1_ref, W2_ref, b2_ref,
          G1_ref, g1_ref, G2_ref, g2_ref, out_ref):
    Bb, V, H = vs_ref.shape
    I = idx_ref.shape[1]
    vs = vs_ref[...]
    idx = idx_ref[...]

    origs = []
    valss = []
    for i in range(I):
        col = idx[:, i]
        acc = jnp.where((col == 0)[:, None], vs[:, 0, :], 0.0)
        for v in range(1, V):
            acc = jnp.where((col == v)[:, None], vs[:, v, :], acc)
        origs.append(acc)
        valss.append(vals_ref[:, i, :])
    orig = jnp.concatenate(origs, axis=0)
    vals = jnp.concatenate(valss, axis=0)

    W1a = W1_ref[:, :H]
    W1b = W1_ref[:, H:]
    h = _gelu(_dot_t(orig, W1a) + _dot_t(vals, W1b) + b1_ref[...])
    enc = _dot_t(h, W2_ref[...]) + b2_ref[...]
    g = _gelu(_dot_t(enc, G1_ref[...]) + g1_ref[...])
    gate = jax.nn.sigmoid(
        jnp.sum(g * G2_ref[...], axis=-1, keepdims=True) + g2_ref[0, 0])
    newv = orig * (1.0 - gate) + vals * gate

    for v in range(V):
        row = vs[:, v, :]
        for i in range(I):
            m = (idx[:, i] == v)[:, None]
            row = jnp.where(m, newv[i * Bb:(i + 1) * Bb, :], row)
        out_ref[:, v, :] = row


@jax.jit
def _run(variable_states, intervention_indices, intervention_values,
         W1, b1, W2, b2, G1, g1, G2, g2):
    B, V, H = variable_states.shape
    I = intervention_indices.shape[1]
    Bb = 64
    grid = (B // Bb,)
    b1r = b1.reshape(1, H)
    b2r = b2.reshape(1, H)
    g1r = g1.reshape(1, H)
    g2r = g2.reshape(1, 1)
    full = lambda *shape: pl.BlockSpec(shape, lambda b: (0,) * len(shape))
    return pl.pallas_call(
        _body,
        grid=grid,
        in_specs=[
            pl.BlockSpec((Bb, I), lambda b: (b, 0)),
            pl.BlockSpec((Bb, V, H), lambda b: (b, 0, 0)),
            pl.BlockSpec((Bb, I, H), lambda b: (b, 0, 0)),
            full(H, 2 * H),
            full(1, H),
            full(H, H),
            full(1, H),
            full(H, H),
            full(1, H),
            full(1, H),
            full(1, 1),
        ],
        out_specs=pl.BlockSpec((Bb, V, H), lambda b: (b, 0, 0)),
        out_shape=jax.ShapeDtypeStruct((B, V, H), jnp.float32),
        compiler_params=pltpu.CompilerParams(
            dimension_semantics=("arbitrary",)),
    )(intervention_indices, variable_states, intervention_values,
      W1, b1r, W2, b2r, G1, g1r, G2, g2r)


def kernel(variable_states, edge_probs, intervention_indices,
           intervention_values, W1, b1, W2, b2, G1, g1, G2, g2):
    del edge_probs
    return _run(variable_states, intervention_indices, intervention_values,
                W1, b1, W2, b2, G1, g1, G2, g2)
